# baseline (speedup 1.0000x reference)
import functools

import jax
import jax.numpy as jnp
from jax import lax
from jax.experimental import pallas as pl
from jax.experimental.pallas import tpu as pltpu
from jax.experimental.pallas import tpu_sc as plsc

N_BLK = 1000

NC = 2
NS = 16
CH = 3200
GB = 128
KBUF = CH + 2 * GB + 32


def _dense_pre(x_ref, fcw_ref, fcb_ref, w1_ref, h_ref, xw1_ref):
    x = x_ref[...]
    h = jnp.maximum(
        lax.dot_general(x, fcw_ref[...], (((1,), (1,)), ((), ())),
                        preferred_element_type=jnp.float32)
        + fcb_ref[...][None, :], 0.0)
    h_ref[...] = h.astype(jnp.bfloat16)
    xw1_ref[...] = jnp.dot(x, w1_ref[...], preferred_element_type=jnp.float32)


def _dense_post(agg2_ref, w2_ref, xw1_ref, out_ref):
    agg = jnp.maximum(agg2_ref[0], agg2_ref[1]).astype(jnp.float32)
    out_ref[...] = xw1_ref[...] + jnp.dot(
        agg, w2_ref[...], preferred_element_type=jnp.float32)


def _make_scatter_max(n, d, e):
    e_sc = e // NC
    nch = e_sc // CH
    npt = ((n // NS) + 7) // 8 * 8
    n_pad = NS * npt
    nfb = d // 16
    mesh = plsc.VectorSubcoreMesh(core_axis_name="c", subcore_axis_name="s")

    @functools.partial(
        pl.kernel, mesh=mesh,
        compiler_params=pltpu.CompilerParams(needs_layout_passes=False, use_tc_tiling_on_sc=False),
        out_type=jax.ShapeDtypeStruct((NC, n_pad, d), jnp.bfloat16),
        scratch_types=[
            pltpu.VMEM((CH,), jnp.int32),
            pltpu.VMEM((CH,), jnp.int32),
            pltpu.VMEM((KBUF,), jnp.int32),
            pltpu.VMEM((KBUF,), jnp.int32),
            pltpu.VMEM((GB, d), jnp.bfloat16),
            pltpu.VMEM((npt + 8, d), jnp.bfloat16),
            pltpu.VMEM_SHARED((n, d), jnp.bfloat16),
        ],
    )
    def scatter_max(h_hbm, src_hbm, trg_hbm, out_hbm,
                    src_v, trg_v, ksrc_v, ktrg_v, hrow_v, agg_v, h_sh):
        cid = lax.axis_index("c")
        sid = lax.axis_index("s")
        lo = sid * npt
        ebase = cid * e_sc
        zero32h = jnp.zeros((32,), jnp.bfloat16)
        zero16i = jnp.zeros((16,), jnp.int32)
        dummy16 = jnp.full((16,), npt, jnp.int32)

        def _z(i, _):
            for fb in range(d // 32):
                agg_v[i, pl.ds(fb * 32, 32)] = zero32h
            return 0
        lax.fori_loop(0, npt + 8, _z, 0)

        def _zk(i, _):
            ktrg_v[pl.ds(i * 16, 16)] = zero16i
            ksrc_v[pl.ds(i * 16, 16)] = dummy16
            return 0
        lax.fori_loop(0, KBUF // 16, _zk, 0)

        @pl.when(sid < 5)
        def _stage():
            pltpu.sync_copy(h_hbm.at[pl.ds(sid * 2000, 2000)],
                            h_sh.at[pl.ds(sid * 2000, 2000)])
        plsc.subcore_barrier()

        def chunk_body(c, _):
            base = ebase + c * CH
            pltpu.sync_copy(src_hbm.at[pl.ds(base, CH)], src_v)
            pltpu.sync_copy(trg_hbm.at[pl.ds(base, CH)], trg_v)

            def scan_body(i, cnt):
                for u in range(4):
                    off = (i * 4 + u) * 16
                    s = src_v[pl.ds(off, 16)]
                    t = trg_v[pl.ds(off, 16)]
                    m = (s >= lo) & (s < lo + npt)
                    run = plsc.cumsum(jnp.where(m, 1, 0))
                    pos = cnt + run - 1
                    plsc.store_scatter(ksrc_v, [pos], s - lo, mask=m)
                    plsc.store_scatter(ktrg_v, [pos], t, mask=m)
                    cnt = cnt + run[15]
                return cnt
            cnt = lax.fori_loop(0, CH // 64, scan_body, 0)

            for k in range(GB // 16 + 1):
                ksrc_v[pl.ds(cnt + k * 16, 16)] = dummy16

            def blk_body(g, _):
                pltpu.sync_copy(h_sh.at[ktrg_v.at[pl.ds(g * GB, GB)]],
                                hrow_v)
                for jb in range(GB // 16):
                    rv = ksrc_v[pl.ds(g * GB + jb * 16, 16)]
                    for k in range(16):
                        r = rv[k]
                        j = jb * 16 + k
                        for fb in range(d // 32):
                            h = hrow_v[j, pl.ds(fb * 32, 32)]
                            a = agg_v[r, pl.ds(fb * 32, 32)]
                            agg_v[r, pl.ds(fb * 32, 32)] = jnp.maximum(a, h)
                return 0
            lax.fori_loop(0, (cnt + GB - 1) // GB, blk_body, 0)
            return 0
        lax.fori_loop(0, nch, chunk_body, 0)

        pltpu.sync_copy(agg_v.at[pl.ds(0, npt)], out_hbm.at[cid, pl.ds(lo, npt)])

    return scatter_max


def kernel(input_matrix, adjacency_coo_matrix, fc_w, fc_b, W):
    n, d_in = input_matrix.shape
    d_hid = fc_w.shape[0]
    d_out = W.shape[1]
    e = adjacency_coo_matrix.shape[1]
    grid = n // N_BLK

    w1 = W[:d_in]
    w2 = W[d_in:]

    h, xw1 = pl.pallas_call(
        _dense_pre,
        grid=(grid,),
        in_specs=[
            pl.BlockSpec((N_BLK, d_in), lambda i: (i, 0)),
            pl.BlockSpec((d_hid, d_in), lambda i: (0, 0)),
            pl.BlockSpec((d_hid,), lambda i: (0,)),
            pl.BlockSpec((d_in, d_out), lambda i: (0, 0)),
        ],
        out_specs=[
            pl.BlockSpec((N_BLK, d_hid), lambda i: (i, 0)),
            pl.BlockSpec((N_BLK, d_out), lambda i: (i, 0)),
        ],
        out_shape=[
            jax.ShapeDtypeStruct((n, d_hid), jnp.bfloat16),
            jax.ShapeDtypeStruct((n, d_out), jnp.float32),
        ],
    )(input_matrix, fc_w, fc_b, w1)

    src = adjacency_coo_matrix[0].astype(jnp.int32)
    trg = adjacency_coo_matrix[1].astype(jnp.int32)

    agg2 = _make_scatter_max(n, d_hid, e)(h, src, trg)

    out = pl.pallas_call(
        _dense_post,
        grid=(grid,),
        in_specs=[
            pl.BlockSpec((NC, N_BLK, d_hid), lambda i: (0, i, 0)),
            pl.BlockSpec((d_hid, d_out), lambda i: (0, 0)),
            pl.BlockSpec((N_BLK, d_out), lambda i: (i, 0)),
        ],
        out_specs=pl.BlockSpec((N_BLK, d_out), lambda i: (i, 0)),
        out_shape=jax.ShapeDtypeStruct((n, d_out), jnp.float32),
    )(agg2, w2, xw1)
    return out


# speedup vs baseline: 1.1409x; 1.1409x over previous
import functools

import jax
import jax.numpy as jnp
from jax import lax
from jax.experimental import pallas as pl
from jax.experimental.pallas import tpu as pltpu
from jax.experimental.pallas import tpu_sc as plsc

N_BLK = 1000

NC = 2
NS = 16
CH = 8000
GB = 128
KBUF = CH + 2 * GB + 32


def _dense_pre(x_ref, fcw_ref, fcb_ref, w1_ref, h_ref, xw1_ref):
    x = x_ref[...]
    h = jnp.maximum(
        lax.dot_general(x, fcw_ref[...], (((1,), (1,)), ((), ())),
                        preferred_element_type=jnp.float32)
        + fcb_ref[...][None, :], 0.0)
    h_ref[...] = h.astype(jnp.bfloat16)
    xw1_ref[...] = jnp.dot(x, w1_ref[...], preferred_element_type=jnp.float32)


def _dense_post(agg2_ref, w2_ref, xw1_ref, out_ref):
    agg = jnp.maximum(agg2_ref[0], agg2_ref[1]).astype(jnp.float32)
    out_ref[...] = xw1_ref[...] + jnp.dot(
        agg, w2_ref[...], preferred_element_type=jnp.float32)


def _make_scatter_max(n, d, e):
    e_sc = e // NC
    nch = e_sc // CH
    npt = ((n // NS) + 7) // 8 * 8
    n_pad = NS * npt
    nfb = d // 16
    mesh = plsc.VectorSubcoreMesh(core_axis_name="c", subcore_axis_name="s")

    @functools.partial(
        pl.kernel, mesh=mesh,
        compiler_params=pltpu.CompilerParams(needs_layout_passes=False, use_tc_tiling_on_sc=False),
        out_type=jax.ShapeDtypeStruct((NC, n_pad, d), jnp.bfloat16),
        scratch_types=[
            pltpu.VMEM((CH,), jnp.int32),
            pltpu.VMEM((CH,), jnp.int32),
            pltpu.VMEM((KBUF,), jnp.int32),
            pltpu.VMEM((KBUF,), jnp.int32),
            pltpu.VMEM((GB, d), jnp.bfloat16),
            pltpu.VMEM((npt + 8, d), jnp.bfloat16),
            pltpu.VMEM_SHARED((n, d), jnp.bfloat16),
        ],
    )
    def scatter_max(h_hbm, src_hbm, trg_hbm, out_hbm,
                    src_v, trg_v, ksrc_v, ktrg_v, hrow_v, agg_v, h_sh):
        cid = lax.axis_index("c")
        sid = lax.axis_index("s")
        lo = sid * npt
        ebase = cid * e_sc
        zero32h = jnp.zeros((32,), jnp.bfloat16)
        zero16i = jnp.zeros((16,), jnp.int32)
        dummy16 = jnp.full((16,), npt, jnp.int32)

        def _z(i, _):
            for fb in range(d // 32):
                agg_v[i, pl.ds(fb * 32, 32)] = zero32h
            return 0
        lax.fori_loop(0, npt + 8, _z, 0)

        def _zk(i, _):
            ktrg_v[pl.ds(i * 16, 16)] = zero16i
            ksrc_v[pl.ds(i * 16, 16)] = dummy16
            return 0
        lax.fori_loop(0, KBUF // 16, _zk, 0)

        @pl.when(sid < 5)
        def _stage():
            pltpu.sync_copy(h_hbm.at[pl.ds(sid * 2000, 2000)],
                            h_sh.at[pl.ds(sid * 2000, 2000)])
        plsc.subcore_barrier()

        def chunk_body(c, _):
            base = ebase + c * CH
            pltpu.sync_copy(src_hbm.at[pl.ds(base, CH)], src_v)
            pltpu.sync_copy(trg_hbm.at[pl.ds(base, CH)], trg_v)

            def scan_body(i, cnt):
                for u in range(4):
                    off = (i * 4 + u) * 16
                    s = src_v[pl.ds(off, 16)]
                    t = trg_v[pl.ds(off, 16)]
                    m = (s >= lo) & (s < lo + npt)
                    run = plsc.cumsum(jnp.where(m, 1, 0))
                    pos = cnt + run - 1
                    plsc.store_scatter(ksrc_v, [pos], s - lo, mask=m)
                    plsc.store_scatter(ktrg_v, [pos], t, mask=m)
                    cnt = cnt + run[15]
                return cnt
            cnt = lax.fori_loop(0, CH // 64, scan_body, 0)

            for k in range(GB // 16 + 1):
                ksrc_v[pl.ds(cnt + k * 16, 16)] = dummy16

            def blk_body(g, _):
                pltpu.sync_copy(h_sh.at[ktrg_v.at[pl.ds(g * GB, GB)]],
                                hrow_v)
                for jb in range(GB // 16):
                    rv = ksrc_v[pl.ds(g * GB + jb * 16, 16)]
                    for k in range(16):
                        r = rv[k]
                        j = jb * 16 + k
                        for fb in range(d // 32):
                            h = hrow_v[j, pl.ds(fb * 32, 32)]
                            a = agg_v[r, pl.ds(fb * 32, 32)]
                            agg_v[r, pl.ds(fb * 32, 32)] = jnp.maximum(a, h)
                return 0
            lax.fori_loop(0, (cnt + GB - 1) // GB, blk_body, 0)
            return 0
        lax.fori_loop(0, nch, chunk_body, 0)

        pltpu.sync_copy(agg_v.at[pl.ds(0, npt)], out_hbm.at[cid, pl.ds(lo, npt)])

    return scatter_max


def kernel(input_matrix, adjacency_coo_matrix, fc_w, fc_b, W):
    n, d_in = input_matrix.shape
    d_hid = fc_w.shape[0]
    d_out = W.shape[1]
    e = adjacency_coo_matrix.shape[1]
    grid = n // N_BLK

    w1 = W[:d_in]
    w2 = W[d_in:]

    h, xw1 = pl.pallas_call(
        _dense_pre,
        grid=(grid,),
        in_specs=[
            pl.BlockSpec((N_BLK, d_in), lambda i: (i, 0)),
            pl.BlockSpec((d_hid, d_in), lambda i: (0, 0)),
            pl.BlockSpec((d_hid,), lambda i: (0,)),
            pl.BlockSpec((d_in, d_out), lambda i: (0, 0)),
        ],
        out_specs=[
            pl.BlockSpec((N_BLK, d_hid), lambda i: (i, 0)),
            pl.BlockSpec((N_BLK, d_out), lambda i: (i, 0)),
        ],
        out_shape=[
            jax.ShapeDtypeStruct((n, d_hid), jnp.bfloat16),
            jax.ShapeDtypeStruct((n, d_out), jnp.float32),
        ],
    )(input_matrix, fc_w, fc_b, w1)

    src = adjacency_coo_matrix[0].astype(jnp.int32)
    trg = adjacency_coo_matrix[1].astype(jnp.int32)

    agg2 = _make_scatter_max(n, d_hid, e)(h, src, trg)

    out = pl.pallas_call(
        _dense_post,
        grid=(grid,),
        in_specs=[
            pl.BlockSpec((NC, N_BLK, d_hid), lambda i: (0, i, 0)),
            pl.BlockSpec((d_hid, d_out), lambda i: (0, 0)),
            pl.BlockSpec((N_BLK, d_out), lambda i: (i, 0)),
        ],
        out_specs=pl.BlockSpec((N_BLK, d_out), lambda i: (i, 0)),
        out_shape=jax.ShapeDtypeStruct((n, d_out), jnp.float32),
    )(agg2, w2, xw1)
    return out


# speedup vs baseline: 1.3976x; 1.2250x over previous
import functools

import jax
import jax.numpy as jnp
from jax import lax
from jax.experimental import pallas as pl
from jax.experimental.pallas import tpu as pltpu
from jax.experimental.pallas import tpu_sc as plsc

N_BLK = 1000

NC = 2
NS = 16
CH = 3200
GB = 128
KBLK = 2
KBUF = CH + 2 * GB + 32


def _dense_pre(x_ref, fcw_ref, fcb_ref, w1_ref, h_ref, xw1_ref):
    x = x_ref[...]
    h = jnp.maximum(
        lax.dot_general(x, fcw_ref[...], (((1,), (1,)), ((), ())),
                        preferred_element_type=jnp.float32)
        + fcb_ref[...][None, :], 0.0)
    h_ref[...] = h.astype(jnp.bfloat16)
    xw1_ref[...] = jnp.dot(x, w1_ref[...], preferred_element_type=jnp.float32)


def _dense_post(agg2_ref, w2_ref, xw1_ref, out_ref):
    agg = jnp.maximum(agg2_ref[0], agg2_ref[1]).astype(jnp.float32)
    out_ref[...] = xw1_ref[...] + jnp.dot(
        agg, w2_ref[...], preferred_element_type=jnp.float32)


def _make_scatter_max(n, d, e):
    e_sc = e // NC
    nch = e_sc // CH
    assert nch % 2 == 0 and CH % 64 == 0
    npt = ((n // NS) + 7) // 8 * 8
    n_pad = NS * npt
    mesh = plsc.VectorSubcoreMesh(core_axis_name="c", subcore_axis_name="s")

    @functools.partial(
        pl.kernel, mesh=mesh,
        compiler_params=pltpu.CompilerParams(
            needs_layout_passes=False, use_tc_tiling_on_sc=False),
        out_type=jax.ShapeDtypeStruct((NC, n_pad, d), jnp.bfloat16),
        scratch_types=[
            [pltpu.VMEM((CH,), jnp.int32)] * 2,
            [pltpu.VMEM((CH,), jnp.int32)] * 2,
            [pltpu.VMEM((KBUF,), jnp.int32)] * 2,
            [pltpu.VMEM((KBUF,), jnp.int32)] * 2,
            [pltpu.VMEM((GB, d), jnp.bfloat16)] * KBLK,
            pltpu.VMEM((npt + 8, d), jnp.bfloat16),
            pltpu.VMEM_SHARED((n, d), jnp.bfloat16),
            [pltpu.SemaphoreType.DMA] * 2,
            [pltpu.SemaphoreType.DMA] * KBLK,
        ],
    )
    def scatter_max(h_hbm, src_hbm, trg_hbm, out_hbm,
                    src_v, trg_v, ksrc_v, ktrg_v, hrow_v, agg_v, h_sh,
                    sem_st, sem_g):
        cid = lax.axis_index("c")
        sid = lax.axis_index("s")
        lo = sid * npt
        ebase = cid * e_sc
        zero32h = jnp.zeros((32,), jnp.bfloat16)
        zero16i = jnp.zeros((16,), jnp.int32)
        dummy16 = jnp.full((16,), npt, jnp.int32)

        def _z(i, _):
            for fb in range(d // 32):
                agg_v[i, pl.ds(fb * 32, 32)] = zero32h
            return 0
        lax.fori_loop(0, npt + 8, _z, 0)

        def _zk(i, _):
            for p in range(2):
                ktrg_v[p][pl.ds(i * 16, 16)] = zero16i
                ksrc_v[p][pl.ds(i * 16, 16)] = dummy16
            return 0
        lax.fori_loop(0, KBUF // 16, _zk, 0)

        @pl.when(sid < 5)
        def _stage():
            pltpu.sync_copy(h_hbm.at[pl.ds(sid * 2000, 2000)],
                            h_sh.at[pl.ds(sid * 2000, 2000)])
        plsc.subcore_barrier()

        def issue_st(c, p):
            base = ebase + c * CH
            pltpu.async_copy(src_hbm.at[pl.ds(base, CH)], src_v[p],
                             sem_st[p])
            pltpu.async_copy(trg_hbm.at[pl.ds(base, CH)], trg_v[p],
                             sem_st[p])

        def wait_st(p):
            pltpu.make_async_copy(src_hbm.at[pl.ds(0, CH)], src_v[p],
                                  sem_st[p]).wait()
            pltpu.make_async_copy(trg_hbm.at[pl.ds(0, CH)], trg_v[p],
                                  sem_st[p]).wait()

        def scan(p):
            def scan_body(i, cnt):
                for u in range(4):
                    off = (i * 4 + u) * 16
                    s = src_v[p][pl.ds(off, 16)]
                    t = trg_v[p][pl.ds(off, 16)]
                    m = (s >= lo) & (s < lo + npt)
                    run = plsc.cumsum(jnp.where(m, 1, 0))
                    pos = cnt + run - 1
                    plsc.store_scatter(ksrc_v[p], [pos], s - lo, mask=m)
                    plsc.store_scatter(ktrg_v[p], [pos], t, mask=m)
                    cnt = cnt + run[15]
                return cnt
            cnt = lax.fori_loop(0, CH // 64, scan_body, 0)
            for k in range(GB // 16 + 1):
                ksrc_v[p][pl.ds(cnt + k * 16, 16)] = dummy16
            return cnt

        def issue_gathers(p, cnt):
            nblk = (cnt + GB - 1) // GB
            for k in range(KBLK):
                @pl.when(k < nblk)
                def _():
                    pltpu.async_copy(
                        h_sh.at[ktrg_v[p].at[pl.ds(k * GB, GB)]],
                        hrow_v[k], sem_g[k])

        def maxblk(p, hrow, base):
            def jb_body(jb, _):
                rv = ksrc_v[p][pl.ds(base + jb * 16, 16)]
                for k in range(16):
                    r = rv[k]
                    for fb in range(d // 32):
                        h = hrow[jb * 16 + k, pl.ds(fb * 32, 32)]
                        a = agg_v[r, pl.ds(fb * 32, 32)]
                        agg_v[r, pl.ds(fb * 32, 32)] = jnp.maximum(a, h)
                return 0
            lax.fori_loop(0, GB // 16, jb_body, 0)

        def drain_max(p, cnt):
            nblk = (cnt + GB - 1) // GB
            for k in range(KBLK):
                @pl.when(k < nblk)
                def _():
                    pltpu.make_async_copy(h_hbm.at[pl.ds(0, GB)],
                                          hrow_v[k], sem_g[k]).wait()
                    maxblk(p, hrow_v[k], k * GB)

            def spill(g, _):
                pltpu.sync_copy(h_sh.at[ktrg_v[p].at[pl.ds(g * GB, GB)]],
                                hrow_v[0])
                maxblk(p, hrow_v[0], g * GB)
                return 0
            lax.fori_loop(KBLK, nblk, spill, 0)

        pltpu.sync_copy(src_hbm.at[pl.ds(ebase, CH)], src_v[0])
        pltpu.sync_copy(trg_hbm.at[pl.ds(ebase, CH)], trg_v[0])
        issue_st(1, 1)
        cnt0 = scan(0)
        issue_gathers(0, cnt0)

        def substep(c, p, cnt_prev):
            wait_st(p)

            @pl.when(c + 1 < nch)
            def _():
                issue_st(c + 1, 1 - p)
            cnt_c = scan(p)
            drain_max(1 - p, cnt_prev)
            issue_gathers(p, cnt_c)
            return cnt_c

        def pair_body(i, cnt_prev):
            cnt_a = substep(2 * i + 1, 1, cnt_prev)
            cnt_b = substep(2 * i + 2, 0, cnt_a)
            return cnt_b

        cnt_last = lax.fori_loop(0, nch // 2 - 1, pair_body, cnt0)
        cnt_final = substep(nch - 1, 1, cnt_last)
        drain_max(1, cnt_final)

        pltpu.sync_copy(agg_v.at[pl.ds(0, npt)],
                        out_hbm.at[cid, pl.ds(lo, npt)])

    return scatter_max


def kernel(input_matrix, adjacency_coo_matrix, fc_w, fc_b, W):
    n, d_in = input_matrix.shape
    d_hid = fc_w.shape[0]
    d_out = W.shape[1]
    e = adjacency_coo_matrix.shape[1]
    grid = n // N_BLK

    w1 = W[:d_in]
    w2 = W[d_in:]

    h, xw1 = pl.pallas_call(
        _dense_pre,
        grid=(grid,),
        in_specs=[
            pl.BlockSpec((N_BLK, d_in), lambda i: (i, 0)),
            pl.BlockSpec((d_hid, d_in), lambda i: (0, 0)),
            pl.BlockSpec((d_hid,), lambda i: (0,)),
            pl.BlockSpec((d_in, d_out), lambda i: (0, 0)),
        ],
        out_specs=[
            pl.BlockSpec((N_BLK, d_hid), lambda i: (i, 0)),
            pl.BlockSpec((N_BLK, d_out), lambda i: (i, 0)),
        ],
        out_shape=[
            jax.ShapeDtypeStruct((n, d_hid), jnp.bfloat16),
            jax.ShapeDtypeStruct((n, d_out), jnp.float32),
        ],
    )(input_matrix, fc_w, fc_b, w1)

    src = adjacency_coo_matrix[0].astype(jnp.int32)
    trg = adjacency_coo_matrix[1].astype(jnp.int32)

    agg2 = _make_scatter_max(n, d_hid, e)(h, src, trg)

    out = pl.pallas_call(
        _dense_post,
        grid=(grid,),
        in_specs=[
            pl.BlockSpec((NC, N_BLK, d_hid), lambda i: (0, i, 0)),
            pl.BlockSpec((d_hid, d_out), lambda i: (0, 0)),
            pl.BlockSpec((N_BLK, d_out), lambda i: (i, 0)),
        ],
        out_specs=pl.BlockSpec((N_BLK, d_out), lambda i: (i, 0)),
        out_shape=jax.ShapeDtypeStruct((n, d_out), jnp.float32),
    )(agg2, w2, xw1)
    return out
